# baseline (device time: 29792 ns/iter reference)
import jax
import jax.numpy as jnp
from jax import lax
from jax.experimental import pallas as pl
from jax.experimental.pallas import tpu as pltpu

N_DEV = 32


def kernel(x, Wg, Wu, Wd):
    m, _ = x.shape
    d = Wd.shape[1]
    chunk = m // N_DEV

    def body(x_ref, wg_ref, wu_ref, wd_ref, out_ref,
             partial_ref, red_ref, rs_buf,
             send_sem1, recv_sem1, send_sem2, recv_sem2):
        my = lax.axis_index("i")

        xv = x_ref[:, :]
        gate = jnp.dot(xv, wg_ref[:, :], preferred_element_type=jnp.float32)
        up = jnp.dot(xv, wu_ref[:, :], preferred_element_type=jnp.float32)
        hidden = gate * (up * jax.nn.sigmoid(up))
        partial_ref[:, :] = jnp.dot(hidden, wd_ref[:, :],
                                    preferred_element_type=jnp.float32)

        sends1 = []
        for off in range(1, N_DEV):
            tgt = lax.rem(my + off, N_DEV)
            rdma = pltpu.make_async_remote_copy(
                src_ref=partial_ref.at[pl.ds(tgt * chunk, chunk), :],
                dst_ref=rs_buf.at[off],
                send_sem=send_sem1.at[off],
                recv_sem=recv_sem1.at[off],
                device_id=(tgt,),
                device_id_type=pl.DeviceIdType.MESH,
            )
            rdma.start()
            sends1.append(rdma)

        for rdma in sends1:
            rdma.wait_recv()
        acc = partial_ref[pl.ds(my * chunk, chunk), :] + jnp.sum(
            rs_buf[1:, :, :], axis=0
        )
        red_ref[:, :] = acc
        out_ref[pl.ds(my * chunk, chunk), :] = acc

        sends2 = []
        for off in range(1, N_DEV):
            tgt = lax.rem(my + off, N_DEV)
            rdma = pltpu.make_async_remote_copy(
                src_ref=red_ref,
                dst_ref=out_ref.at[pl.ds(my * chunk, chunk), :],
                send_sem=send_sem2.at[off],
                recv_sem=recv_sem2.at[off],
                device_id=(tgt,),
                device_id_type=pl.DeviceIdType.MESH,
            )
            rdma.start()
            sends2.append(rdma)

        for off in range(1, N_DEV):
            src = lax.rem(my - off + N_DEV, N_DEV)
            recv = pltpu.make_async_remote_copy(
                src_ref=red_ref,
                dst_ref=out_ref.at[pl.ds(src * chunk, chunk), :],
                send_sem=send_sem2.at[off],
                recv_sem=recv_sem2.at[off],
                device_id=(src,),
                device_id_type=pl.DeviceIdType.MESH,
            )
            recv.wait_recv()

        for rdma in sends1:
            rdma.wait_send()
        for rdma in sends2:
            rdma.wait_send()

    return pl.pallas_call(
        body,
        out_shape=jax.ShapeDtypeStruct((m, d), jnp.float32),
        in_specs=[pl.BlockSpec(memory_space=pltpu.VMEM)] * 4,
        out_specs=pl.BlockSpec(memory_space=pltpu.VMEM),
        scratch_shapes=[
            pltpu.VMEM((m, d), jnp.float32),
            pltpu.VMEM((chunk, d), jnp.float32),
            pltpu.VMEM((N_DEV, chunk, d), jnp.float32),
            pltpu.SemaphoreType.DMA((N_DEV,)),
            pltpu.SemaphoreType.DMA((N_DEV,)),
            pltpu.SemaphoreType.DMA((N_DEV,)),
            pltpu.SemaphoreType.DMA((N_DEV,)),
        ],
    )(x, Wg, Wu, Wd)


# device time: 25759 ns/iter; 1.1566x vs baseline; 1.1566x over previous
import jax
import jax.numpy as jnp
from jax import lax
from jax.experimental import pallas as pl
from jax.experimental.pallas import tpu as pltpu

N_DEV = 32


def kernel(x, Wg, Wu, Wd):
    m, _ = x.shape
    d = Wd.shape[1]
    chunk = m // N_DEV

    def body(x_ref, wg_ref, wu_ref, wd_ref, out_ref,
             partial_ref, red_ref, rs_buf,
             send_sem1, recv_sem1, send_sem2, recv_sem2):
        my = lax.axis_index("i")

        barrier_sem = pltpu.get_barrier_semaphore()
        for off in range(1, N_DEV):
            tgt = lax.rem(my + off, N_DEV)
            pl.semaphore_signal(
                barrier_sem, inc=1,
                device_id=(tgt,), device_id_type=pl.DeviceIdType.MESH,
            )

        xv = x_ref[:, :]
        gate = jnp.dot(xv, wg_ref[:, :], preferred_element_type=jnp.float32)
        up = jnp.dot(xv, wu_ref[:, :], preferred_element_type=jnp.float32)
        hidden = gate * (up * jax.nn.sigmoid(up))
        partial_ref[:, :] = jnp.dot(hidden, wd_ref[:, :],
                                    preferred_element_type=jnp.float32)

        pl.semaphore_wait(barrier_sem, N_DEV - 1)

        sends1 = []
        for off in range(1, N_DEV):
            tgt = lax.rem(my + off, N_DEV)
            rdma = pltpu.make_async_remote_copy(
                src_ref=partial_ref.at[pl.ds(tgt * chunk, chunk), :],
                dst_ref=rs_buf.at[off],
                send_sem=send_sem1.at[off],
                recv_sem=recv_sem1.at[off],
                device_id=(tgt,),
                device_id_type=pl.DeviceIdType.MESH,
            )
            rdma.start()
            sends1.append(rdma)

        for rdma in sends1:
            rdma.wait_recv()
        acc = partial_ref[pl.ds(my * chunk, chunk), :] + jnp.sum(
            rs_buf[1:, :, :], axis=0
        )
        red_ref[:, :] = acc
        out_ref[pl.ds(my * chunk, chunk), :] = acc

        sends2 = []
        for off in range(1, N_DEV):
            tgt = lax.rem(my + off, N_DEV)
            rdma = pltpu.make_async_remote_copy(
                src_ref=red_ref,
                dst_ref=out_ref.at[pl.ds(my * chunk, chunk), :],
                send_sem=send_sem2.at[off],
                recv_sem=recv_sem2.at[off],
                device_id=(tgt,),
                device_id_type=pl.DeviceIdType.MESH,
            )
            rdma.start()
            sends2.append(rdma)

        for off in range(1, N_DEV):
            src = lax.rem(my - off + N_DEV, N_DEV)
            recv = pltpu.make_async_remote_copy(
                src_ref=red_ref,
                dst_ref=out_ref.at[pl.ds(src * chunk, chunk), :],
                send_sem=send_sem2.at[off],
                recv_sem=recv_sem2.at[off],
                device_id=(src,),
                device_id_type=pl.DeviceIdType.MESH,
            )
            recv.wait_recv()

        for rdma in sends1:
            rdma.wait_send()
        for rdma in sends2:
            rdma.wait_send()

    return pl.pallas_call(
        body,
        out_shape=jax.ShapeDtypeStruct((m, d), jnp.float32),
        in_specs=[pl.BlockSpec(memory_space=pltpu.VMEM)] * 4,
        out_specs=pl.BlockSpec(memory_space=pltpu.VMEM),
        scratch_shapes=[
            pltpu.VMEM((m, d), jnp.float32),
            pltpu.VMEM((chunk, d), jnp.float32),
            pltpu.VMEM((N_DEV, chunk, d), jnp.float32),
            pltpu.SemaphoreType.DMA((N_DEV,)),
            pltpu.SemaphoreType.DMA((N_DEV,)),
            pltpu.SemaphoreType.DMA((N_DEV,)),
            pltpu.SemaphoreType.DMA((N_DEV,)),
        ],
        compiler_params=pltpu.CompilerParams(collective_id=0),
    )(x, Wg, Wu, Wd)


# device time: 20364 ns/iter; 1.4630x vs baseline; 1.2649x over previous
import jax
import jax.numpy as jnp
from jax import lax
from jax.experimental import pallas as pl
from jax.experimental.pallas import tpu as pltpu

N_DEV = 32


def kernel(x, Wg, Wu, Wd):
    m, _ = x.shape
    d = Wd.shape[1]
    chunk = m // N_DEV

    def body(x_ref, wg_ref, wu_ref, wd_ref, out_ref,
             partial_ref, red_ref, rs_buf,
             send_sem1, recv_sem1, send_sem2, recv_sem2, local_sem):
        my = lax.axis_index("i")

        barrier_sem = pltpu.get_barrier_semaphore()
        pl.semaphore_signal(barrier_sem, inc=1)
        pl.semaphore_wait(barrier_sem, 1)

        xb = x_ref[:, :].astype(jnp.bfloat16)
        wgb = wg_ref[:, :].astype(jnp.bfloat16)
        wub = wu_ref[:, :].astype(jnp.bfloat16)
        wdb = wd_ref[:, :].astype(jnp.bfloat16)

        n_groups = 4
        rows_per = m // n_groups
        per_g = N_DEV // n_groups
        for g in range(n_groups):
            r0 = g * rows_per
            xg = xb[r0:r0 + rows_per, :]
            gate = jnp.dot(xg, wgb, preferred_element_type=jnp.float32)
            up = jnp.dot(xg, wub, preferred_element_type=jnp.float32)
            hid = (gate * (up * jax.nn.sigmoid(up))).astype(jnp.bfloat16)
            partial_ref[pl.ds(r0, rows_per), :] = jnp.dot(
                hid, wdb, preferred_element_type=jnp.float32
            ).astype(jnp.bfloat16)
            for j in range(g * per_g, (g + 1) * per_g):
                off = lax.rem(j - my + N_DEV, N_DEV)

                @pl.when(off != 0)
                def _send(j=j, off=off):
                    pltpu.make_async_remote_copy(
                        src_ref=partial_ref.at[pl.ds(j * chunk, chunk), :],
                        dst_ref=rs_buf.at[off],
                        send_sem=send_sem1.at[off],
                        recv_sem=recv_sem1.at[off],
                        device_id=(j,),
                        device_id_type=pl.DeviceIdType.MESH,
                    ).start()

        sends1 = []
        for off in range(1, N_DEV):
            rdma = pltpu.make_async_remote_copy(
                src_ref=partial_ref.at[pl.ds(0, chunk), :],
                dst_ref=rs_buf.at[off],
                send_sem=send_sem1.at[off],
                recv_sem=recv_sem1.at[off],
                device_id=(0,),
                device_id_type=pl.DeviceIdType.MESH,
            )
            rdma.wait_recv()
            sends1.append(rdma)
        acc = partial_ref[pl.ds(my * chunk, chunk), :].astype(
            jnp.float32
        ) + jnp.sum(rs_buf[1:, :, :].astype(jnp.float32), axis=0)
        red_ref[:, :] = acc
        local_copy = pltpu.make_async_copy(
            red_ref, out_ref.at[pl.ds(my * chunk, chunk), :], local_sem
        )
        local_copy.start()

        sends2 = []
        for off in range(1, N_DEV):
            tgt = lax.rem(my + off, N_DEV)
            rdma = pltpu.make_async_remote_copy(
                src_ref=red_ref,
                dst_ref=out_ref.at[pl.ds(my * chunk, chunk), :],
                send_sem=send_sem2.at[off],
                recv_sem=recv_sem2.at[off],
                device_id=(tgt,),
                device_id_type=pl.DeviceIdType.MESH,
            )
            rdma.start()
            sends2.append(rdma)

        for off in range(1, N_DEV):
            src = lax.rem(my - off + N_DEV, N_DEV)
            recv = pltpu.make_async_remote_copy(
                src_ref=red_ref,
                dst_ref=out_ref.at[pl.ds(src * chunk, chunk), :],
                send_sem=send_sem2.at[off],
                recv_sem=recv_sem2.at[off],
                device_id=(src,),
                device_id_type=pl.DeviceIdType.MESH,
            )
            recv.wait_recv()

        local_copy.wait()
        for rdma in sends1:
            rdma.wait_send()
        for rdma in sends2:
            rdma.wait_send()

    return pl.pallas_call(
        body,
        out_shape=jax.ShapeDtypeStruct((m, d), jnp.float32),
        in_specs=[pl.BlockSpec(memory_space=pltpu.VMEM)] * 4,
        out_specs=pl.BlockSpec(memory_space=pl.ANY),
        scratch_shapes=[
            pltpu.VMEM((m, d), jnp.bfloat16),
            pltpu.VMEM((chunk, d), jnp.float32),
            pltpu.VMEM((N_DEV, chunk, d), jnp.bfloat16),
            pltpu.SemaphoreType.DMA((N_DEV,)),
            pltpu.SemaphoreType.DMA((N_DEV,)),
            pltpu.SemaphoreType.DMA((N_DEV,)),
            pltpu.SemaphoreType.DMA((N_DEV,)),
            pltpu.SemaphoreType.DMA,
        ],
        compiler_params=pltpu.CompilerParams(collective_id=0),
    )(x, Wg, Wu, Wd)
